# Initial kernel scaffold; baseline (speedup 1.0000x reference)
import jax
import jax.numpy as jnp
from jax.experimental import pallas as pl


def kernel(pos1, pos2, W1, b1, W2):
    raise NotImplementedError("write your pallas kernel here")


# speedup vs baseline: 1.0688x; 1.0688x over previous
import jax
import jax.numpy as jnp
from jax.experimental import pallas as pl
from jax.experimental.pallas import tpu as pltpu

EMB = 1024
TB = 1024


def _mlp_block(x_ref, w1_ref, b1_ref, w2_ref, out_ref):
    x = x_ref[...]
    x = jnp.where(jnp.abs(x) < 1e-06, 0.0, x)
    p1 = x[:, 0:1]
    p2 = x[:, 1:2]
    h = p1 * w1_ref[0:1, :] + p2 * w1_ref[1:2, :] + b1_ref[...]
    h = h * jax.nn.sigmoid(h)
    y = jnp.dot(h, w2_ref[...], preferred_element_type=jnp.float32)
    out_ref[...] = 16.0 * jax.nn.sigmoid(y)


def kernel(pos1, pos2, W1, b1, W2):
    B, S = pos1.shape
    n = B * S
    x = jnp.stack((pos1.reshape(n), pos2.reshape(n)), axis=-1)
    grid = n // TB
    out = pl.pallas_call(
        _mlp_block,
        grid=(grid,),
        in_specs=[
            pl.BlockSpec((TB, 2), lambda i: (i, 0)),
            pl.BlockSpec((2, EMB), lambda i: (0, 0)),
            pl.BlockSpec((1, EMB), lambda i: (0, 0)),
            pl.BlockSpec((EMB, EMB), lambda i: (0, 0)),
        ],
        out_specs=pl.BlockSpec((TB, EMB), lambda i: (i, 0)),
        out_shape=jax.ShapeDtypeStruct((n, EMB), jnp.float32),
        compiler_params=pltpu.CompilerParams(
            dimension_semantics=("parallel",),
        ),
    )(x, W1, b1.reshape(1, EMB), W2)
    return out.reshape(B, S, EMB)
